# baseline (speedup 1.0000x reference)
import functools

import jax
import jax.numpy as jnp
from jax.experimental import pallas as pl
from jax.experimental.pallas import tpu as pltpu

N_NODES = 10000
N_EDGES = 320000
D = 128
HEADS = 8
K = 2500


def _proj_kernel(x_ref, wl_ref, wr_ref, ol_ref, or_ref):
    x = jnp.nan_to_num(x_ref[...], nan=0.0)
    ol_ref[...] = jnp.dot(x, wl_ref[...], preferred_element_type=jnp.float32)
    or_ref[...] = jnp.dot(x, wr_ref[...], preferred_element_type=jnp.float32)


def _pool_projections(x, Wl_pool, Wr_pool):
    blk = 1000
    grid = (N_NODES // blk,)
    out = pl.pallas_call(
        _proj_kernel,
        grid=grid,
        in_specs=[
            pl.BlockSpec((blk, D), lambda i: (i, 0)),
            pl.BlockSpec((D, D), lambda i: (0, 0)),
            pl.BlockSpec((D, D), lambda i: (0, 0)),
        ],
        out_specs=[
            pl.BlockSpec((blk, D), lambda i: (i, 0)),
            pl.BlockSpec((blk, D), lambda i: (i, 0)),
        ],
        out_shape=[
            jax.ShapeDtypeStruct((N_NODES, D), jnp.float32),
            jax.ShapeDtypeStruct((N_NODES, D), jnp.float32),
        ],
    )(x, Wl_pool.T, Wr_pool.T)
    return out


def _assign_scores(x, edge_index, Wl_assign, Wr_assign, att_assign):
    x = jnp.nan_to_num(x, nan=0.0)
    al = (x @ Wl_assign.T)[:, 0]
    ar = (x @ Wr_assign.T)[:, 0]
    src = edge_index[0]
    dst = edge_index[1]
    xj = jnp.take(al, src, axis=0)
    xi = jnp.take(ar, dst, axis=0)
    alpha = jax.nn.leaky_relu(xi + xj, negative_slope=0.2) * att_assign[0, 0, 0]
    amax = jax.ops.segment_max(alpha, dst, num_segments=N_NODES)
    amax = jnp.where(jnp.isfinite(amax), amax, 0.0)
    ex = jnp.exp(alpha - jnp.take(amax, dst, axis=0))
    denom = jax.ops.segment_sum(ex, dst, num_segments=N_NODES)
    a2 = ex / (jnp.take(denom, dst, axis=0) + 1e-16)
    scores = jax.ops.segment_sum(xj * a2, dst, num_segments=N_NODES)
    return scores


def _pool_conv(xl, xr, edge_index, att_pool):
    C = D // HEADS
    src = edge_index[0]
    dst = edge_index[1]
    xlh = xl.reshape(N_NODES, HEADS, C)
    xrh = xr.reshape(N_NODES, HEADS, C)
    xj = jnp.take(xlh, src, axis=0)
    xi = jnp.take(xrh, dst, axis=0)
    alpha = (jax.nn.leaky_relu(xi + xj, negative_slope=0.2) * att_pool).sum(axis=-1)
    amax = jax.ops.segment_max(alpha, dst, num_segments=N_NODES)
    amax = jnp.where(jnp.isfinite(amax), amax, 0.0)
    ex = jnp.exp(alpha - jnp.take(amax, dst, axis=0))
    denom = jax.ops.segment_sum(ex, dst, num_segments=N_NODES)
    a2 = ex / (jnp.take(denom, dst, axis=0) + 1e-16)
    msg = xj * a2[..., None]
    out = jax.ops.segment_sum(msg, dst, num_segments=N_NODES)
    return out.reshape(N_NODES, D)


def kernel(x, edge_index, batch, Wl_pool, Wr_pool, att_pool, Wl_assign, Wr_assign, att_assign):
    scores = _assign_scores(x, edge_index, Wl_assign, Wr_assign, att_assign)
    p = jax.nn.softmax(scores, axis=0)
    _, top_idx = jax.lax.top_k(p, K)
    xl, xr = _pool_projections(x, Wl_pool, Wr_pool)
    x_pooled = _pool_conv(xl, xr, edge_index, att_pool)
    x_coarse = jnp.take(x_pooled, top_idx, axis=0)
    batch_coarse = jnp.take(batch, top_idx, axis=0)
    return (x_coarse, batch_coarse)


# speedup vs baseline: 5.5084x; 5.5084x over previous
import functools

import jax
import jax.numpy as jnp
from jax import lax
from jax.experimental import pallas as pl
from jax.experimental.pallas import tpu as pltpu
from jax.experimental.pallas import tpu_sc as plsc

N_NODES = 10000
N_EDGES = 320000
D = 128
HEADS = 8
C = D // HEADS
K = 2500
NW = 32
CHUNK = 128


def _proj_kernel(x_ref, wl_ref, wr_ref, wa_ref, ol_ref, or_ref, oa_ref):
    x = jnp.nan_to_num(x_ref[...], nan=0.0)
    ol_ref[...] = jnp.dot(x, wl_ref[...], preferred_element_type=jnp.float32)
    or_ref[...] = jnp.dot(x, wr_ref[...], preferred_element_type=jnp.float32)
    oa_ref[...] = jnp.dot(x, wa_ref[...], preferred_element_type=jnp.float32)


def _projections(x, WlT, WrT, WaT):
    blk = 1000
    return pl.pallas_call(
        _proj_kernel,
        grid=(N_NODES // blk,),
        in_specs=[
            pl.BlockSpec((blk, D), lambda i: (i, 0)),
            pl.BlockSpec((D, D), lambda i: (0, 0)),
            pl.BlockSpec((D, D), lambda i: (0, 0)),
            pl.BlockSpec((D, D), lambda i: (0, 0)),
        ],
        out_specs=[
            pl.BlockSpec((blk, D), lambda i: (i, 0)),
            pl.BlockSpec((blk, D), lambda i: (i, 0)),
            pl.BlockSpec((blk, D), lambda i: (i, 0)),
        ],
        out_shape=[
            jax.ShapeDtypeStruct((N_NODES, D), jnp.float32),
            jax.ShapeDtypeStruct((N_NODES, D), jnp.float32),
            jax.ShapeDtypeStruct((N_NODES, D), jnp.float32),
        ],
    )(x, WlT, WrT, WaT)


_MESH = plsc.VectorSubcoreMesh(core_axis_name="c", subcore_axis_name="s")
_NCH = N_EDGES // CHUNK
_ITERS = (_NCH + NW - 1) // NW


@functools.partial(
    pl.kernel,
    mesh=_MESH,
    out_type=[
        jax.ShapeDtypeStruct((N_EDGES, D), jnp.float32),
        jax.ShapeDtypeStruct((N_EDGES, D), jnp.float32),
    ],
    scratch_types=[
        pltpu.VMEM((CHUNK,), jnp.int32),
        pltpu.VMEM((CHUNK,), jnp.int32),
        pltpu.VMEM((CHUNK, D), jnp.float32),
        pltpu.VMEM((CHUNK, D), jnp.float32),
        pltpu.SemaphoreType.DMA,
        pltpu.SemaphoreType.DMA,
    ],
)
def _gather_rows_kernel(xl_hbm, xr_hbm, src_hbm, dst_hbm, oj_hbm, oi_hbm,
                        idxs_v, idxd_v, rows_a, rows_b, sema, semb):
    wid = lax.axis_index("s") * 2 + lax.axis_index("c")

    def body(i, carry):
        ch = i * NW + wid

        @pl.when(ch < _NCH)
        def _():
            base = ch * CHUNK
            pltpu.sync_copy(src_hbm.at[pl.ds(base, CHUNK)], idxs_v)
            pltpu.sync_copy(dst_hbm.at[pl.ds(base, CHUNK)], idxd_v)
            cpa = pltpu.async_copy(xl_hbm.at[idxs_v], rows_a, sema)
            cpb = pltpu.async_copy(xr_hbm.at[idxd_v], rows_b, semb)
            cpa.wait()
            cpb.wait()
            pltpu.sync_copy(rows_a, oj_hbm.at[pl.ds(base, CHUNK)])
            pltpu.sync_copy(rows_b, oi_hbm.at[pl.ds(base, CHUNK)])

        return carry

    lax.fori_loop(0, _ITERS, body, 0)


def _edge_math_kernel(xj_ref, xi_ref, dstm_ref, att_ref, rep_ref, tile_ref,
                      exp_ref, om_ref, od_ref):
    xj = xj_ref[...]
    xi = xi_ref[...]
    t = xi + xj
    t = jnp.where(t >= 0, t, 0.2 * t)
    logit = jnp.dot(t, att_ref[...], preferred_element_type=jnp.float32)
    w = jnp.exp(logit)
    wfull = jnp.dot(w, rep_ref[...], preferred_element_type=jnp.float32)
    om_ref[...] = xj * wfull
    blk = w.shape[0]
    slots = jax.lax.broadcasted_iota(jnp.int32, (blk, 16), 1)
    oh = (dstm_ref[...] == slots).astype(jnp.float32)
    oh_rep = jnp.dot(oh, exp_ref[...], preferred_element_type=jnp.float32)
    w_tile = jnp.dot(w, tile_ref[...], preferred_element_type=jnp.float32)
    od_ref[...] = oh_rep * w_tile


def _edge_math(xj, xi, dst_mod16, att_mat, rep_mat, tile_mat, exp_mat):
    blk = 2000
    return pl.pallas_call(
        _edge_math_kernel,
        grid=(N_EDGES // blk,),
        in_specs=[
            pl.BlockSpec((blk, D), lambda i: (i, 0)),
            pl.BlockSpec((blk, D), lambda i: (i, 0)),
            pl.BlockSpec((blk, 1), lambda i: (i, 0)),
            pl.BlockSpec((D, HEADS), lambda i: (0, 0)),
            pl.BlockSpec((HEADS, D), lambda i: (0, 0)),
            pl.BlockSpec((HEADS, D), lambda i: (0, 0)),
            pl.BlockSpec((16, D), lambda i: (0, 0)),
        ],
        out_specs=[
            pl.BlockSpec((blk, D), lambda i: (i, 0)),
            pl.BlockSpec((blk, D), lambda i: (i, 0)),
        ],
        out_shape=[
            jax.ShapeDtypeStruct((N_EDGES, D), jnp.float32),
            jax.ShapeDtypeStruct((N_EDGES, D), jnp.float32),
        ],
    )(xj, xi, dst_mod16, att_mat, rep_mat, tile_mat, exp_mat)


_ROWS_PER_TILE = 624
_TAIL_ROWS = N_NODES - 16 * _ROWS_PER_TILE
_CH_PER_CORE = _NCH // 2
_SC_ITERS = (_CH_PER_CORE + 15) // 16


_N_DEN = 640


@functools.partial(
    pl.kernel,
    mesh=_MESH,
    out_type=[
        jax.ShapeDtypeStruct((2, N_NODES, D), jnp.float32),
        jax.ShapeDtypeStruct((2, _N_DEN, D), jnp.float32),
    ],
    scratch_types=[
        pltpu.VMEM((CHUNK,), jnp.int32),
        pltpu.VMEM((CHUNK,), jnp.int32),
        pltpu.VMEM((CHUNK, D), jnp.float32),
        pltpu.VMEM((CHUNK, D), jnp.float32),
        pltpu.VMEM_SHARED((N_NODES, D), jnp.float32),
        pltpu.VMEM_SHARED((_N_DEN, D), jnp.float32),
    ],
)
def _scatter_add_kernel(msg_hbm, den_hbm, dst_hbm, dst16_hbm, zeros_hbm,
                        outm_hbm, outd_hbm,
                        idx_v, idx16_v, rows_v, denrows_v, accm_sh, accd_sh):
    cid = lax.axis_index("c")
    sid = lax.axis_index("s")
    r0 = sid * _ROWS_PER_TILE
    pltpu.sync_copy(zeros_hbm.at[pl.ds(r0, _ROWS_PER_TILE)],
                    accm_sh.at[pl.ds(r0, _ROWS_PER_TILE)])

    @pl.when(sid == 15)
    def _zero_tail():
        pltpu.sync_copy(zeros_hbm.at[pl.ds(16 * _ROWS_PER_TILE, _TAIL_ROWS)],
                        accm_sh.at[pl.ds(16 * _ROWS_PER_TILE, _TAIL_ROWS)])

    @pl.when(sid == 0)
    def _zero_den():
        pltpu.sync_copy(zeros_hbm.at[pl.ds(0, _N_DEN)], accd_sh)

    plsc.subcore_barrier()

    def body(j, carry):
        ch = cid * _CH_PER_CORE + j * 16 + sid

        @pl.when(ch < (cid + 1) * _CH_PER_CORE)
        def _():
            base = ch * CHUNK
            pltpu.sync_copy(dst_hbm.at[pl.ds(base, CHUNK)], idx_v)
            pltpu.sync_copy(dst16_hbm.at[pl.ds(base, CHUNK)], idx16_v)
            pltpu.sync_copy(msg_hbm.at[pl.ds(base, CHUNK)], rows_v)
            pltpu.sync_copy(den_hbm.at[pl.ds(base, CHUNK)], denrows_v)
            pltpu.sync_copy(rows_v, accm_sh.at[idx_v], add=True)
            pltpu.sync_copy(denrows_v, accd_sh.at[idx16_v], add=True)

        return carry

    lax.fori_loop(0, _SC_ITERS, body, 0)
    plsc.subcore_barrier()
    pltpu.sync_copy(accm_sh.at[pl.ds(r0, _ROWS_PER_TILE)],
                    outm_hbm.at[cid, pl.ds(r0, _ROWS_PER_TILE)])

    @pl.when(sid == 15)
    def _out_tail():
        pltpu.sync_copy(accm_sh.at[pl.ds(16 * _ROWS_PER_TILE, _TAIL_ROWS)],
                        outm_hbm.at[cid, pl.ds(16 * _ROWS_PER_TILE, _TAIL_ROWS)])

    @pl.when(sid == 1)
    def _out_den():
        pltpu.sync_copy(accd_sh, outd_hbm.at[cid])


def _merge_kernel(accm_ref, accd_ref, rep_ref, o_ref):
    num = accm_ref[0] + accm_ref[1]
    den = accd_ref[0] + accd_ref[1]
    denfull = jnp.dot(den, rep_ref[...], preferred_element_type=jnp.float32)
    o_ref[...] = num / (denfull + 1e-16)


def _merge(accm, accd, rep_mat):
    blk = 1000
    return pl.pallas_call(
        _merge_kernel,
        grid=(N_NODES // blk,),
        in_specs=[
            pl.BlockSpec((2, blk, D), lambda i: (0, i, 0)),
            pl.BlockSpec((2, blk, HEADS), lambda i: (0, i, 0)),
            pl.BlockSpec((HEADS, D), lambda i: (0, 0)),
        ],
        out_specs=pl.BlockSpec((blk, D), lambda i: (i, 0)),
        out_shape=jax.ShapeDtypeStruct((N_NODES, D), jnp.float32),
    )(accm, accd, rep_mat)


_NEG_INF = float("-inf")


@functools.partial(
    pl.kernel,
    mesh=_MESH,
    out_type=[
        jax.ShapeDtypeStruct((N_EDGES,), jnp.float32),
        jax.ShapeDtypeStruct((NW, N_NODES), jnp.float32),
    ],
    scratch_types=[
        pltpu.VMEM((N_NODES,), jnp.float32),
        pltpu.VMEM((N_NODES,), jnp.float32),
        pltpu.VMEM((N_NODES,), jnp.float32),
        pltpu.VMEM((CHUNK,), jnp.int32),
        pltpu.VMEM((CHUNK,), jnp.int32),
        pltpu.VMEM((CHUNK,), jnp.float32),
        pltpu.VMEM((16,), jnp.float32),
    ],
)
def _assign_pass1_kernel(al_hbm, ar_hbm, src_hbm, dst_hbm, att_hbm,
                         alpha_hbm, amax_hbm,
                         al_t, ar_t, amax_t, srcb, dstb, ab, attb):
    wid = lax.axis_index("s") * 2 + lax.axis_index("c")
    pltpu.sync_copy(al_hbm, al_t)
    pltpu.sync_copy(ar_hbm, ar_t)
    pltpu.sync_copy(att_hbm, attb)
    attv = attb[...]
    ninf = jnp.full((16,), _NEG_INF, jnp.float32)

    def initb(i, c):
        amax_t[pl.ds(i * 16, 16)] = ninf
        return c

    lax.fori_loop(0, N_NODES // 16, initb, 0)

    def body(i, carry):
        ch = i * NW + wid

        @pl.when(ch < _NCH)
        def _():
            base = ch * CHUNK
            pltpu.sync_copy(src_hbm.at[pl.ds(base, CHUNK)], srcb)
            pltpu.sync_copy(dst_hbm.at[pl.ds(base, CHUNK)], dstb)
            for j in range(CHUNK // 16):
                s16 = srcb[pl.ds(j * 16, 16)]
                d16 = dstb[pl.ds(j * 16, 16)]
                alv = plsc.load_gather(al_t, [s16])
                arv = plsc.load_gather(ar_t, [d16])
                t = arv + alv
                a = jnp.where(t >= 0, t, jnp.float32(0.2) * t) * attv
                ab[pl.ds(j * 16, 16)] = a

            def upd(jj, c):
                d0 = dstb[jj]
                av = ab[jj]
                cur = amax_t[d0]
                amax_t[d0] = jnp.maximum(cur, av)
                return c

            lax.fori_loop(0, CHUNK, upd, 0)
            pltpu.sync_copy(ab, alpha_hbm.at[pl.ds(base, CHUNK)])

        return carry

    lax.fori_loop(0, _ITERS, body, 0)
    pltpu.sync_copy(amax_t, amax_hbm.at[wid])


def _amax_merge_kernel(am_ref, o_ref):
    m = jnp.max(am_ref[...], axis=0, keepdims=True)
    o_ref[...] = jnp.where(jnp.isfinite(m), m, 0.0)


def _amax_merge(am32):
    return pl.pallas_call(
        _amax_merge_kernel,
        in_specs=[pl.BlockSpec((NW, N_NODES), lambda: (0, 0))],
        out_specs=pl.BlockSpec((1, N_NODES), lambda: (0, 0)),
        out_shape=jax.ShapeDtypeStruct((1, N_NODES), jnp.float32),
    )(am32)


@functools.partial(
    pl.kernel,
    mesh=_MESH,
    out_type=jax.ShapeDtypeStruct((N_EDGES,), jnp.float32),
    scratch_types=[
        pltpu.VMEM((N_NODES,), jnp.float32),
        pltpu.VMEM((CHUNK,), jnp.int32),
        pltpu.VMEM((CHUNK,), jnp.float32),
    ],
)
def _assign_pass2_kernel(alpha_hbm, dst_hbm, amax_hbm, ex_hbm, am_t, dstb, ab):
    wid = lax.axis_index("s") * 2 + lax.axis_index("c")
    pltpu.sync_copy(amax_hbm, am_t)

    def body(i, carry):
        ch = i * NW + wid

        @pl.when(ch < _NCH)
        def _():
            base = ch * CHUNK
            pltpu.sync_copy(dst_hbm.at[pl.ds(base, CHUNK)], dstb)
            pltpu.sync_copy(alpha_hbm.at[pl.ds(base, CHUNK)], ab)
            for j in range(CHUNK // 16):
                d16 = dstb[pl.ds(j * 16, 16)]
                am = plsc.load_gather(am_t, [d16])
                ab[pl.ds(j * 16, 16)] = jnp.exp(ab[pl.ds(j * 16, 16)] - am)
            pltpu.sync_copy(ab, ex_hbm.at[pl.ds(base, CHUNK)])

        return carry

    lax.fori_loop(0, _ITERS, body, 0)


@functools.partial(
    pl.kernel,
    mesh=_MESH,
    out_type=jax.ShapeDtypeStruct((N_EDGES,), jnp.float32),
    scratch_types=[
        pltpu.VMEM((N_NODES,), jnp.float32),
        pltpu.VMEM((N_NODES,), jnp.float32),
        pltpu.VMEM((CHUNK,), jnp.int32),
        pltpu.VMEM((CHUNK,), jnp.int32),
        pltpu.VMEM((CHUNK,), jnp.float32),
    ],
)
def _assign_pass3_kernel(ex_hbm, src_hbm, dst_hbm, al_hbm, den_hbm, co_hbm,
                         al_t, den_t, srcb, dstb, eb):
    wid = lax.axis_index("s") * 2 + lax.axis_index("c")
    pltpu.sync_copy(al_hbm, al_t)
    pltpu.sync_copy(den_hbm, den_t)

    def body(i, carry):
        ch = i * NW + wid

        @pl.when(ch < _NCH)
        def _():
            base = ch * CHUNK
            pltpu.sync_copy(src_hbm.at[pl.ds(base, CHUNK)], srcb)
            pltpu.sync_copy(dst_hbm.at[pl.ds(base, CHUNK)], dstb)
            pltpu.sync_copy(ex_hbm.at[pl.ds(base, CHUNK)], eb)
            for j in range(CHUNK // 16):
                s16 = srcb[pl.ds(j * 16, 16)]
                d16 = dstb[pl.ds(j * 16, 16)]
                xj = plsc.load_gather(al_t, [s16])
                dn = plsc.load_gather(den_t, [d16])
                a2 = eb[pl.ds(j * 16, 16)] / (dn + jnp.float32(1e-16))
                eb[pl.ds(j * 16, 16)] = xj * a2
            pltpu.sync_copy(eb, co_hbm.at[pl.ds(base, CHUNK)])

        return carry

    lax.fori_loop(0, _ITERS, body, 0)


def kernel(x, edge_index, batch, Wl_pool, Wr_pool, att_pool, Wl_assign, Wr_assign, att_assign):
    WaT = jnp.zeros((D, D), jnp.float32)
    WaT = WaT.at[:, 0].set(Wl_assign[0])
    WaT = WaT.at[:, 1].set(Wr_assign[0])

    xl, xr, aux = _projections(x, Wl_pool.T, Wr_pool.T, WaT)
    al = aux[:, 0]
    ar = aux[:, 1]

    src = edge_index[0]
    dst = edge_index[1]

    xj_a = jnp.take(al, src, axis=0)
    xi_a = jnp.take(ar, dst, axis=0)
    alpha = jax.nn.leaky_relu(xi_a + xj_a, negative_slope=0.2) * att_assign[0, 0, 0]
    amax = jax.ops.segment_max(alpha, dst, num_segments=N_NODES)
    amax = jnp.where(jnp.isfinite(amax), amax, 0.0)
    ex = jnp.exp(alpha - jnp.take(amax, dst, axis=0))
    denom = jax.ops.segment_sum(ex, dst, num_segments=N_NODES)
    a2 = ex / (jnp.take(denom, dst, axis=0) + 1e-16)
    scores = jax.ops.segment_sum(xj_a * a2, dst, num_segments=N_NODES)
    p = jax.nn.softmax(scores, axis=0)
    _, top_idx = jax.lax.top_k(p, K)

    xj_rows, xi_rows = _gather_rows_kernel(xl, xr, src, dst)

    att_mat = jnp.zeros((D, HEADS), jnp.float32)
    ar_idx = jnp.arange(D)
    att_mat = att_mat.at[ar_idx, ar_idx // C].set(att_pool.reshape(D))
    rep_mat = jnp.zeros((HEADS, D), jnp.float32)
    rep_mat = rep_mat.at[ar_idx // C, ar_idx].set(1.0)

    tile_mat = jnp.tile(jnp.eye(HEADS, dtype=jnp.float32), (1, 16))
    exp_mat = jnp.repeat(jnp.eye(16, dtype=jnp.float32), HEADS, axis=1)
    dst_mod16 = (dst % 16).astype(jnp.int32).reshape(N_EDGES, 1)
    dst_div16 = (dst // 16).astype(jnp.int32)

    msg, denrow = _edge_math(xj_rows, xi_rows, dst_mod16, att_mat, rep_mat,
                             tile_mat, exp_mat)
    zeros_acc = jnp.zeros((N_NODES, D), jnp.float32)
    accm, accd = _scatter_add_kernel(msg, denrow, dst, dst_div16, zeros_acc)
    accd2 = accd[:, :N_NODES // 16].reshape(2, N_NODES, HEADS)
    x_pooled = _merge(accm, accd2, rep_mat)

    x_coarse = jnp.take(x_pooled, top_idx, axis=0)
    batch_coarse = jnp.take(batch, top_idx, axis=0)
    return (x_coarse, batch_coarse)


# speedup vs baseline: 26.2303x; 4.7618x over previous
import functools

import jax
import jax.numpy as jnp
from jax import lax
from jax.experimental import pallas as pl
from jax.experimental.pallas import tpu as pltpu
from jax.experimental.pallas import tpu_sc as plsc

N_NODES = 10000
N_EDGES = 320000
D = 128
HEADS = 8
C = D // HEADS
K = 2500
NW = 32
CHUNK = 128


def _proj_kernel(x_ref, wl_ref, wr_ref, wa_ref, ol_ref, or_ref, oa_ref):
    x = jnp.nan_to_num(x_ref[...], nan=0.0)
    ol_ref[...] = jnp.dot(x, wl_ref[...], preferred_element_type=jnp.float32)
    or_ref[...] = jnp.dot(x, wr_ref[...], preferred_element_type=jnp.float32)
    oa_ref[...] = jnp.dot(x, wa_ref[...], preferred_element_type=jnp.float32)


def _projections(x, WlT, WrT, WaT):
    blk = 1000
    return pl.pallas_call(
        _proj_kernel,
        grid=(N_NODES // blk,),
        in_specs=[
            pl.BlockSpec((blk, D), lambda i: (i, 0)),
            pl.BlockSpec((D, D), lambda i: (0, 0)),
            pl.BlockSpec((D, D), lambda i: (0, 0)),
            pl.BlockSpec((D, D), lambda i: (0, 0)),
        ],
        out_specs=[
            pl.BlockSpec((blk, D), lambda i: (i, 0)),
            pl.BlockSpec((blk, D), lambda i: (i, 0)),
            pl.BlockSpec((blk, D), lambda i: (i, 0)),
        ],
        out_shape=[
            jax.ShapeDtypeStruct((N_NODES, D), jnp.float32),
            jax.ShapeDtypeStruct((N_NODES, D), jnp.float32),
            jax.ShapeDtypeStruct((N_NODES, D), jnp.float32),
        ],
    )(x, WlT, WrT, WaT)


_MESH = plsc.VectorSubcoreMesh(core_axis_name="c", subcore_axis_name="s")
_NCH = N_EDGES // CHUNK
_ITERS = (_NCH + NW - 1) // NW


@functools.partial(
    pl.kernel,
    mesh=_MESH,
    out_type=[
        jax.ShapeDtypeStruct((N_EDGES, D), jnp.float32),
        jax.ShapeDtypeStruct((N_EDGES, D), jnp.float32),
    ],
    scratch_types=[
        pltpu.VMEM((CHUNK,), jnp.int32),
        pltpu.VMEM((CHUNK,), jnp.int32),
        pltpu.VMEM((CHUNK, D), jnp.float32),
        pltpu.VMEM((CHUNK, D), jnp.float32),
        pltpu.SemaphoreType.DMA,
        pltpu.SemaphoreType.DMA,
    ],
)
def _gather_rows_kernel(xl_hbm, xr_hbm, src_hbm, dst_hbm, oj_hbm, oi_hbm,
                        idxs_v, idxd_v, rows_a, rows_b, sema, semb):
    wid = lax.axis_index("s") * 2 + lax.axis_index("c")

    def body(i, carry):
        ch = i * NW + wid

        @pl.when(ch < _NCH)
        def _():
            base = ch * CHUNK
            pltpu.sync_copy(src_hbm.at[pl.ds(base, CHUNK)], idxs_v)
            pltpu.sync_copy(dst_hbm.at[pl.ds(base, CHUNK)], idxd_v)
            cpa = pltpu.async_copy(xl_hbm.at[idxs_v], rows_a, sema)
            cpb = pltpu.async_copy(xr_hbm.at[idxd_v], rows_b, semb)
            cpa.wait()
            cpb.wait()
            pltpu.sync_copy(rows_a, oj_hbm.at[pl.ds(base, CHUNK)])
            pltpu.sync_copy(rows_b, oi_hbm.at[pl.ds(base, CHUNK)])

        return carry

    lax.fori_loop(0, _ITERS, body, 0)


def _edge_math_kernel(xj_ref, xi_ref, dstm_ref, att_ref, rep_ref, tile_ref,
                      exp_ref, om_ref, od_ref):
    xj = xj_ref[...]
    xi = xi_ref[...]
    t = xi + xj
    t = jnp.where(t >= 0, t, 0.2 * t)
    logit = jnp.dot(t, att_ref[...], preferred_element_type=jnp.float32)
    w = jnp.exp(logit)
    wfull = jnp.dot(w, rep_ref[...], preferred_element_type=jnp.float32)
    om_ref[...] = xj * wfull
    blk = w.shape[0]
    slots = jax.lax.broadcasted_iota(jnp.int32, (blk, 16), 1)
    oh = (dstm_ref[...] == slots).astype(jnp.float32)
    oh_rep = jnp.dot(oh, exp_ref[...], preferred_element_type=jnp.float32)
    w_tile = jnp.dot(w, tile_ref[...], preferred_element_type=jnp.float32)
    od_ref[...] = oh_rep * w_tile


def _edge_math(xj, xi, dst_mod16, att_mat, rep_mat, tile_mat, exp_mat):
    blk = 2000
    return pl.pallas_call(
        _edge_math_kernel,
        grid=(N_EDGES // blk,),
        in_specs=[
            pl.BlockSpec((blk, D), lambda i: (i, 0)),
            pl.BlockSpec((blk, D), lambda i: (i, 0)),
            pl.BlockSpec((blk, 1), lambda i: (i, 0)),
            pl.BlockSpec((D, HEADS), lambda i: (0, 0)),
            pl.BlockSpec((HEADS, D), lambda i: (0, 0)),
            pl.BlockSpec((HEADS, D), lambda i: (0, 0)),
            pl.BlockSpec((16, D), lambda i: (0, 0)),
        ],
        out_specs=[
            pl.BlockSpec((blk, D), lambda i: (i, 0)),
            pl.BlockSpec((blk, D), lambda i: (i, 0)),
        ],
        out_shape=[
            jax.ShapeDtypeStruct((N_EDGES, D), jnp.float32),
            jax.ShapeDtypeStruct((N_EDGES, D), jnp.float32),
        ],
    )(xj, xi, dst_mod16, att_mat, rep_mat, tile_mat, exp_mat)


_ROWS_PER_TILE = 624
_TAIL_ROWS = N_NODES - 16 * _ROWS_PER_TILE
_CH_PER_CORE = _NCH // 2
_SC_ITERS = (_CH_PER_CORE + 15) // 16


_N_DEN = 640


@functools.partial(
    pl.kernel,
    mesh=_MESH,
    out_type=[
        jax.ShapeDtypeStruct((2, N_NODES, D), jnp.float32),
        jax.ShapeDtypeStruct((2, _N_DEN, D), jnp.float32),
    ],
    scratch_types=[
        pltpu.VMEM((CHUNK,), jnp.int32),
        pltpu.VMEM((CHUNK,), jnp.int32),
        pltpu.VMEM((CHUNK, D), jnp.float32),
        pltpu.VMEM((CHUNK, D), jnp.float32),
        pltpu.VMEM_SHARED((N_NODES, D), jnp.float32),
        pltpu.VMEM_SHARED((_N_DEN, D), jnp.float32),
    ],
)
def _scatter_add_kernel(msg_hbm, den_hbm, dst_hbm, dst16_hbm, zeros_hbm,
                        outm_hbm, outd_hbm,
                        idx_v, idx16_v, rows_v, denrows_v, accm_sh, accd_sh):
    cid = lax.axis_index("c")
    sid = lax.axis_index("s")
    r0 = sid * _ROWS_PER_TILE
    pltpu.sync_copy(zeros_hbm.at[pl.ds(r0, _ROWS_PER_TILE)],
                    accm_sh.at[pl.ds(r0, _ROWS_PER_TILE)])

    @pl.when(sid == 15)
    def _zero_tail():
        pltpu.sync_copy(zeros_hbm.at[pl.ds(16 * _ROWS_PER_TILE, _TAIL_ROWS)],
                        accm_sh.at[pl.ds(16 * _ROWS_PER_TILE, _TAIL_ROWS)])

    @pl.when(sid == 0)
    def _zero_den():
        pltpu.sync_copy(zeros_hbm.at[pl.ds(0, _N_DEN)], accd_sh)

    plsc.subcore_barrier()

    def body(j, carry):
        ch = cid * _CH_PER_CORE + j * 16 + sid

        @pl.when(ch < (cid + 1) * _CH_PER_CORE)
        def _():
            base = ch * CHUNK
            pltpu.sync_copy(dst_hbm.at[pl.ds(base, CHUNK)], idx_v)
            pltpu.sync_copy(dst16_hbm.at[pl.ds(base, CHUNK)], idx16_v)
            pltpu.sync_copy(msg_hbm.at[pl.ds(base, CHUNK)], rows_v)
            pltpu.sync_copy(den_hbm.at[pl.ds(base, CHUNK)], denrows_v)
            pltpu.sync_copy(rows_v, accm_sh.at[idx_v], add=True)
            pltpu.sync_copy(denrows_v, accd_sh.at[idx16_v], add=True)

        return carry

    lax.fori_loop(0, _SC_ITERS, body, 0)
    plsc.subcore_barrier()
    pltpu.sync_copy(accm_sh.at[pl.ds(r0, _ROWS_PER_TILE)],
                    outm_hbm.at[cid, pl.ds(r0, _ROWS_PER_TILE)])

    @pl.when(sid == 15)
    def _out_tail():
        pltpu.sync_copy(accm_sh.at[pl.ds(16 * _ROWS_PER_TILE, _TAIL_ROWS)],
                        outm_hbm.at[cid, pl.ds(16 * _ROWS_PER_TILE, _TAIL_ROWS)])

    @pl.when(sid == 1)
    def _out_den():
        pltpu.sync_copy(accd_sh, outd_hbm.at[cid])


def _merge_kernel(accm_ref, accd_ref, rep_ref, o_ref):
    num = accm_ref[0] + accm_ref[1]
    den = accd_ref[0] + accd_ref[1]
    denfull = jnp.dot(den, rep_ref[...], preferred_element_type=jnp.float32)
    o_ref[...] = num / (denfull + 1e-16)


def _merge(accm, accd, rep_mat):
    blk = 1000
    return pl.pallas_call(
        _merge_kernel,
        grid=(N_NODES // blk,),
        in_specs=[
            pl.BlockSpec((2, blk, D), lambda i: (0, i, 0)),
            pl.BlockSpec((2, blk, HEADS), lambda i: (0, i, 0)),
            pl.BlockSpec((HEADS, D), lambda i: (0, 0)),
        ],
        out_specs=pl.BlockSpec((blk, D), lambda i: (i, 0)),
        out_shape=jax.ShapeDtypeStruct((N_NODES, D), jnp.float32),
    )(accm, accd, rep_mat)


_NEG_INF = float("-inf")
_PIB = lax.GatherScatterMode.PROMISE_IN_BOUNDS


@functools.partial(
    pl.kernel,
    compiler_params=pltpu.CompilerParams(needs_layout_passes=False),
    mesh=_MESH,
    out_type=[
        jax.ShapeDtypeStruct((N_EDGES,), jnp.float32),
        jax.ShapeDtypeStruct((NW, N_NODES), jnp.float32),
    ],
    scratch_types=[
        pltpu.VMEM((N_NODES,), jnp.float32),
        pltpu.VMEM((N_NODES,), jnp.float32),
        pltpu.VMEM((N_NODES,), jnp.float32),
        pltpu.VMEM((CHUNK,), jnp.int32),
        pltpu.VMEM((CHUNK,), jnp.int32),
        pltpu.VMEM((CHUNK,), jnp.float32),
        pltpu.VMEM((16,), jnp.float32),
        pltpu.VMEM((16,), jnp.int32),
        pltpu.VMEM((16,), jnp.float32),
    ],
)
def _assign_pass1_kernel(al_hbm, ar_hbm, src_hbm, dst_hbm, att_hbm,
                         alpha_hbm, amax_hbm,
                         al_t, ar_t, amax_t, srcb, dstb, ab, attb, kbuf, vbuf):
    wid = lax.axis_index("s") * 2 + lax.axis_index("c")
    pltpu.sync_copy(al_hbm, al_t)
    pltpu.sync_copy(ar_hbm, ar_t)
    pltpu.sync_copy(att_hbm, attb)
    attv = attb[...]
    ninf = jnp.full((16,), _NEG_INF, jnp.float32)

    def initb(i, c):
        amax_t[pl.ds(i * 16, 16)] = ninf
        return c

    lax.fori_loop(0, N_NODES // 16, initb, 0)

    def body(i, carry):
        ch = i * NW + wid

        @pl.when(ch < _NCH)
        def _():
            base = ch * CHUNK
            pltpu.sync_copy(src_hbm.at[pl.ds(base, CHUNK)], srcb)
            pltpu.sync_copy(dst_hbm.at[pl.ds(base, CHUNK)], dstb)
            iota16 = lax.iota(jnp.int32, 16)
            for j in range(CHUNK // 16):
                s16 = srcb[pl.ds(j * 16, 16)]
                d16 = dstb[pl.ds(j * 16, 16)]
                alv = plsc.load_gather(al_t, [s16])
                arv = plsc.load_gather(ar_t, [d16])
                t = arv + alv
                a = jnp.where(t >= 0, t, jnp.float32(0.2) * t) * attv
                ab[pl.ds(j * 16, 16)] = a
                ks, vs = plsc.sort_key_val(d16, a)
                kbuf[...] = ks
                for dsh in (1, 2, 4, 8):
                    vbuf[...] = vs
                    idx = jnp.maximum(iota16 - dsh, 0)
                    kp = plsc.load_gather(kbuf, [idx])
                    vp = plsc.load_gather(vbuf, [idx])
                    same = (kp == ks) & (iota16 >= dsh)
                    vs = jnp.where(same, jnp.maximum(vs, vp), vs)
                nxt = plsc.load_gather(kbuf, [jnp.minimum(iota16 + 1, 15)])
                is_last = (nxt != ks) | (iota16 == 15)
                cur = plsc.load_gather(amax_t, [ks])
                plsc.store_scatter(amax_t, [ks], jnp.maximum(cur, vs),
                                   mask=is_last)
            pltpu.sync_copy(ab, alpha_hbm.at[pl.ds(base, CHUNK)])

        return carry

    lax.fori_loop(0, _ITERS, body, 0)
    pltpu.sync_copy(amax_t, amax_hbm.at[wid])


def _amax_merge_kernel(am_ref, o_ref):
    m = jnp.max(am_ref[...], axis=0, keepdims=True)
    o_ref[...] = jnp.where(jnp.isfinite(m), m, 0.0)


def _amax_merge(am32):
    return pl.pallas_call(
        _amax_merge_kernel,
        in_specs=[pl.BlockSpec((NW, N_NODES), lambda: (0, 0))],
        out_specs=pl.BlockSpec((1, N_NODES), lambda: (0, 0)),
        out_shape=jax.ShapeDtypeStruct((1, N_NODES), jnp.float32),
    )(am32)


@functools.partial(
    pl.kernel,
    compiler_params=pltpu.CompilerParams(needs_layout_passes=False),
    mesh=_MESH,
    out_type=jax.ShapeDtypeStruct((N_EDGES,), jnp.float32),
    scratch_types=[
        pltpu.VMEM((N_NODES,), jnp.float32),
        pltpu.VMEM((CHUNK,), jnp.int32),
        pltpu.VMEM((CHUNK,), jnp.float32),
    ],
)
def _assign_pass2_kernel(alpha_hbm, dst_hbm, amax_hbm, ex_hbm, am_t, dstb, ab):
    wid = lax.axis_index("s") * 2 + lax.axis_index("c")
    pltpu.sync_copy(amax_hbm, am_t)

    def body(i, carry):
        ch = i * NW + wid

        @pl.when(ch < _NCH)
        def _():
            base = ch * CHUNK
            pltpu.sync_copy(dst_hbm.at[pl.ds(base, CHUNK)], dstb)
            pltpu.sync_copy(alpha_hbm.at[pl.ds(base, CHUNK)], ab)
            for j in range(CHUNK // 16):
                d16 = dstb[pl.ds(j * 16, 16)]
                am = plsc.load_gather(am_t, [d16])
                ab[pl.ds(j * 16, 16)] = jnp.exp(ab[pl.ds(j * 16, 16)] - am)
            pltpu.sync_copy(ab, ex_hbm.at[pl.ds(base, CHUNK)])

        return carry

    lax.fori_loop(0, _ITERS, body, 0)


@functools.partial(
    pl.kernel,
    compiler_params=pltpu.CompilerParams(needs_layout_passes=False),
    mesh=_MESH,
    out_type=jax.ShapeDtypeStruct((N_EDGES,), jnp.float32),
    scratch_types=[
        pltpu.VMEM((N_NODES,), jnp.float32),
        pltpu.VMEM((N_NODES,), jnp.float32),
        pltpu.VMEM((CHUNK,), jnp.int32),
        pltpu.VMEM((CHUNK,), jnp.int32),
        pltpu.VMEM((CHUNK,), jnp.float32),
    ],
)
def _assign_pass3_kernel(ex_hbm, src_hbm, dst_hbm, al_hbm, den_hbm, co_hbm,
                         al_t, den_t, srcb, dstb, eb):
    wid = lax.axis_index("s") * 2 + lax.axis_index("c")
    pltpu.sync_copy(al_hbm, al_t)
    pltpu.sync_copy(den_hbm, den_t)

    def body(i, carry):
        ch = i * NW + wid

        @pl.when(ch < _NCH)
        def _():
            base = ch * CHUNK
            pltpu.sync_copy(src_hbm.at[pl.ds(base, CHUNK)], srcb)
            pltpu.sync_copy(dst_hbm.at[pl.ds(base, CHUNK)], dstb)
            pltpu.sync_copy(ex_hbm.at[pl.ds(base, CHUNK)], eb)
            for j in range(CHUNK // 16):
                s16 = srcb[pl.ds(j * 16, 16)]
                d16 = dstb[pl.ds(j * 16, 16)]
                xj = plsc.load_gather(al_t, [s16])
                dn = plsc.load_gather(den_t, [d16])
                a2 = eb[pl.ds(j * 16, 16)] / (dn + jnp.float32(1e-16))
                eb[pl.ds(j * 16, 16)] = xj * a2
            pltpu.sync_copy(eb, co_hbm.at[pl.ds(base, CHUNK)])

        return carry

    lax.fori_loop(0, _ITERS, body, 0)


def kernel(x, edge_index, batch, Wl_pool, Wr_pool, att_pool, Wl_assign, Wr_assign, att_assign):
    WaT = jnp.zeros((D, D), jnp.float32)
    WaT = WaT.at[:, 0].set(Wl_assign[0])
    WaT = WaT.at[:, 1].set(Wr_assign[0])

    xl, xr, aux = _projections(x, Wl_pool.T, Wr_pool.T, WaT)
    al = aux[:, 0]
    ar = aux[:, 1]

    src = edge_index[0]
    dst = edge_index[1]

    att_vec = jnp.full((16,), att_assign[0, 0, 0], jnp.float32)
    alpha, amax32 = _assign_pass1_kernel(al, ar, src, dst, att_vec)
    amax = _amax_merge(amax32)[0]
    ex = _assign_pass2_kernel(alpha, dst, amax)
    denom = jax.ops.segment_sum(ex, dst, num_segments=N_NODES)
    contrib = _assign_pass3_kernel(ex, src, dst, al, denom)
    scores = jax.ops.segment_sum(contrib, dst, num_segments=N_NODES)
    p = jax.nn.softmax(scores, axis=0)
    _, top_idx = jax.lax.top_k(p, K)

    xj_rows, xi_rows = _gather_rows_kernel(xl, xr, src, dst)

    att_mat = jnp.zeros((D, HEADS), jnp.float32)
    ar_idx = jnp.arange(D)
    att_mat = att_mat.at[ar_idx, ar_idx // C].set(att_pool.reshape(D))
    rep_mat = jnp.zeros((HEADS, D), jnp.float32)
    rep_mat = rep_mat.at[ar_idx // C, ar_idx].set(1.0)

    tile_mat = jnp.tile(jnp.eye(HEADS, dtype=jnp.float32), (1, 16))
    exp_mat = jnp.repeat(jnp.eye(16, dtype=jnp.float32), HEADS, axis=1)
    dst_mod16 = (dst % 16).astype(jnp.int32).reshape(N_EDGES, 1)
    dst_div16 = (dst // 16).astype(jnp.int32)

    msg, denrow = _edge_math(xj_rows, xi_rows, dst_mod16, att_mat, rep_mat,
                             tile_mat, exp_mat)
    zeros_acc = jnp.zeros((N_NODES, D), jnp.float32)
    accm, accd = _scatter_add_kernel(msg, denrow, dst, dst_div16, zeros_acc)
    accd2 = accd[:, :N_NODES // 16].reshape(2, N_NODES, HEADS)
    x_pooled = _merge(accm, accd2, rep_mat)

    x_coarse = jnp.take(x_pooled, top_idx, axis=0)
    batch_coarse = jnp.take(batch, top_idx, axis=0)
    return (x_coarse, batch_coarse)
